# baseline (speedup 1.0000x reference)
import jax
import jax.numpy as jnp
from jax.experimental import pallas as pl
from jax.experimental.pallas import tpu as pltpu

N_LAYER = 24
N_HEAD = 16
HEAD_SIZE = 64
TOTAL_DIM = 3072
OUT_DIM = TOTAL_DIM // 3
T = 4096
T_CHUNKS = 16
HALF = N_LAYER // 2


def _body(e_ref, w1_ref, w2_ref, b_ref, o_ref, vec_ref):
    t = pl.program_id(0)

    @pl.when(t == 0)
    def _init():
        vec_ref[...] = jnp.zeros_like(vec_ref)

    @pl.when(t < T_CHUNKS)
    def _mean():
        vec_ref[...] += jnp.sum(e_ref[...], axis=0, keepdims=True) * (1.0 / T)

    @pl.when(t >= T_CHUNKS)
    def _matvec():
        v = vec_ref[...]
        q = t - T_CHUNKS

        @pl.when(q % 2 == 0)
        def _even():
            o_ref[0] = (
                jax.lax.dot_general(v, w1_ref[0, 0], (((1,), (1,)), ((), ())),
                                    preferred_element_type=jnp.float32)
                + b_ref[0]
            )

        @pl.when(q % 2 == 1)
        def _odd():
            o_ref[0] = (
                jax.lax.dot_general(v, w2_ref[0, 0], (((1,), (1,)), ((), ())),
                                    preferred_element_type=jnp.float32)
                + b_ref[0]
            )


def kernel(system_emb, W_proj, b_proj):
    e = system_emb.reshape(T, TOTAL_DIM)
    wr = W_proj.reshape(HALF, 2, OUT_DIM, TOTAL_DIM)
    b3 = b_proj.reshape(N_LAYER, 1, OUT_DIM)

    def _w1(t):
        q = jnp.maximum(t - T_CHUNKS, 0)
        return (q // 2, 0, 0, 0)

    def _w2(t):
        q = jnp.maximum(t - T_CHUNKS, 0)
        return (q // 2, 1, 0, 0)

    def _bo(t):
        return (jnp.maximum(t - T_CHUNKS, 0), 0, 0)

    out = pl.pallas_call(
        _body,
        grid=(T_CHUNKS + N_LAYER,),
        in_specs=[
            pl.BlockSpec((T // T_CHUNKS, TOTAL_DIM),
                         lambda t: (jnp.minimum(t, T_CHUNKS - 1), 0)),
            pl.BlockSpec((1, 1, OUT_DIM, TOTAL_DIM), _w1),
            pl.BlockSpec((1, 1, OUT_DIM, TOTAL_DIM), _w2),
            pl.BlockSpec((1, 1, OUT_DIM), _bo),
        ],
        out_specs=pl.BlockSpec((1, 1, OUT_DIM), _bo),
        out_shape=jax.ShapeDtypeStruct((N_LAYER, 1, OUT_DIM), jnp.float32),
        scratch_shapes=[pltpu.VMEM((1, TOTAL_DIM), jnp.float32)],
    )(e, wr, wr, b3)
    return out.reshape(N_LAYER, N_HEAD, HEAD_SIZE)


# speedup vs baseline: 1.2406x; 1.2406x over previous
import jax
import jax.numpy as jnp
from jax.experimental import pallas as pl
from jax.experimental.pallas import tpu as pltpu

N_LAYER = 24
N_HEAD = 16
HEAD_SIZE = 64
TOTAL_DIM = 3072
OUT_DIM = TOTAL_DIM // 3
T = 4096
T_CHUNKS = 16
LPS = 2
STEPS = N_LAYER // LPS


def _body(e_ref, w_ref, b_ref, o_ref, vec_ref):
    t = pl.program_id(0)

    @pl.when(t == 0)
    def _init():
        vec_ref[...] = jnp.zeros_like(vec_ref)

    @pl.when(t < T_CHUNKS)
    def _mean():
        vec_ref[...] += jnp.sum(e_ref[...], axis=0, keepdims=True) * (1.0 / T)

    @pl.when(t >= T_CHUNKS)
    def _matvec():
        v = vec_ref[...]
        r = jax.lax.dot_general(
            v, w_ref[...], (((1,), (2,)), ((), ())),
            preferred_element_type=jnp.float32,
        )
        o_ref[...] = r.reshape(LPS, 1, OUT_DIM) + b_ref[...]


def kernel(system_emb, W_proj, b_proj):
    e = system_emb.reshape(T, TOTAL_DIM)
    b3 = b_proj.reshape(N_LAYER, 1, OUT_DIM)

    def _wm(t):
        return (jnp.maximum(t - T_CHUNKS, 0), 0, 0)

    out = pl.pallas_call(
        _body,
        grid=(T_CHUNKS + STEPS,),
        in_specs=[
            pl.BlockSpec((T // T_CHUNKS, TOTAL_DIM),
                         lambda t: (jnp.minimum(t, T_CHUNKS - 1), 0)),
            pl.BlockSpec((LPS, OUT_DIM, TOTAL_DIM), _wm),
            pl.BlockSpec((LPS, 1, OUT_DIM), _wm),
        ],
        out_specs=pl.BlockSpec((LPS, 1, OUT_DIM), _wm),
        out_shape=jax.ShapeDtypeStruct((N_LAYER, 1, OUT_DIM), jnp.float32),
        scratch_shapes=[pltpu.VMEM((1, TOTAL_DIM), jnp.float32)],
    )(e, W_proj, b3)
    return out.reshape(N_LAYER, N_HEAD, HEAD_SIZE)


# speedup vs baseline: 1.2623x; 1.0174x over previous
import jax
import jax.numpy as jnp
from jax.experimental import pallas as pl
from jax.experimental.pallas import tpu as pltpu

N_LAYER = 24
N_HEAD = 16
HEAD_SIZE = 64
TOTAL_DIM = 3072
OUT_DIM = TOTAL_DIM // 3
T = 4096
T_CHUNKS = 16


def _fused_body(e_ref, w_ref, b_ref, out_ref, vec_ref):
    t = pl.program_id(0)

    @pl.when(t == 0)
    def _init():
        vec_ref[...] = jnp.zeros_like(vec_ref)

    @pl.when(t < T_CHUNKS)
    def _mean():
        vec_ref[...] += jnp.sum(e_ref[...], axis=0, keepdims=True) * (1.0 / T)

    @pl.when(t >= T_CHUNKS)
    def _matvec():
        out_ref[0] = (
            jax.lax.dot_general(
                vec_ref[...], w_ref[0], (((1,), (1,)), ((), ())),
                preferred_element_type=jnp.float32,
            )
            + b_ref[0]
        )


def kernel(system_emb, W_proj, b_proj):
    e = system_emb.reshape(T, TOTAL_DIM)
    out = pl.pallas_call(
        _fused_body,
        grid=(T_CHUNKS + N_LAYER,),
        in_specs=[
            pl.BlockSpec(
                (T // T_CHUNKS, TOTAL_DIM),
                lambda t: (jnp.minimum(t, T_CHUNKS - 1), 0),
            ),
            pl.BlockSpec(
                (1, OUT_DIM, TOTAL_DIM),
                lambda t: (jnp.maximum(t - T_CHUNKS, 0), 0, 0),
            ),
            pl.BlockSpec(
                (1, 1, OUT_DIM),
                lambda t: (jnp.maximum(t - T_CHUNKS, 0), 0, 0),
            ),
        ],
        out_specs=pl.BlockSpec(
            (1, 1, OUT_DIM),
            lambda t: (jnp.maximum(t - T_CHUNKS, 0), 0, 0),
        ),
        out_shape=jax.ShapeDtypeStruct((N_LAYER, 1, OUT_DIM), jnp.float32),
        scratch_shapes=[pltpu.VMEM((1, TOTAL_DIM), jnp.float32)],
    )(e, W_proj, b_proj.reshape(N_LAYER, 1, OUT_DIM))
    return out.reshape(N_LAYER, N_HEAD, HEAD_SIZE)


# speedup vs baseline: 1.2923x; 1.0238x over previous
import jax
import jax.numpy as jnp
from jax.experimental import pallas as pl
from jax.experimental.pallas import tpu as pltpu

N_LAYER = 24
N_HEAD = 16
HEAD_SIZE = 64
TOTAL_DIM = 3072
OUT_DIM = TOTAL_DIM // 3
T = 4096
T_CHUNKS = 4


def _fused_body(e_ref, w_ref, b_ref, out_ref, vec_ref):
    t = pl.program_id(0)

    @pl.when(t == 0)
    def _init():
        vec_ref[...] = jnp.zeros_like(vec_ref)

    @pl.when(t < T_CHUNKS)
    def _mean():
        vec_ref[...] += jnp.sum(e_ref[...], axis=0, keepdims=True) * (1.0 / T)

    @pl.when(t >= T_CHUNKS)
    def _matvec():
        out_ref[0] = (
            jax.lax.dot_general(
                vec_ref[...], w_ref[0], (((1,), (1,)), ((), ())),
                preferred_element_type=jnp.float32,
            )
            + b_ref[0]
        )


def kernel(system_emb, W_proj, b_proj):
    e = system_emb.reshape(T, TOTAL_DIM)
    out = pl.pallas_call(
        _fused_body,
        grid=(T_CHUNKS + N_LAYER,),
        in_specs=[
            pl.BlockSpec(
                (T // T_CHUNKS, TOTAL_DIM),
                lambda t: (jnp.minimum(t, T_CHUNKS - 1), 0),
            ),
            pl.BlockSpec(
                (1, OUT_DIM, TOTAL_DIM),
                lambda t: (jnp.maximum(t - T_CHUNKS, 0), 0, 0),
            ),
            pl.BlockSpec(
                (1, 1, OUT_DIM),
                lambda t: (jnp.maximum(t - T_CHUNKS, 0), 0, 0),
            ),
        ],
        out_specs=pl.BlockSpec(
            (1, 1, OUT_DIM),
            lambda t: (jnp.maximum(t - T_CHUNKS, 0), 0, 0),
        ),
        out_shape=jax.ShapeDtypeStruct((N_LAYER, 1, OUT_DIM), jnp.float32),
        scratch_shapes=[pltpu.VMEM((1, TOTAL_DIM), jnp.float32)],
    )(e, W_proj, b_proj.reshape(N_LAYER, 1, OUT_DIM))
    return out.reshape(N_LAYER, N_HEAD, HEAD_SIZE)
